# baseline (speedup 1.0000x reference)
import functools

import jax
import jax.numpy as jnp
from jax import lax
from jax.experimental import pallas as pl
from jax.experimental.pallas import tpu as pltpu
from jax.experimental.pallas import tpu_sc as plsc

_B = 16384
_DIM = 32
_N = 1000000
_NC = 2
_NS = 16
_NW = _NC * _NS
_SLAB = 244 * 128
_CWL = 256
_NCHUNK = 124
_TAILR = _NW * _SLAB + _CWL * 2
_TAILR = 999936
_TAILW = _N - _TAILR
_NBUCK = _NCHUNK
_BCAP = 40
_LCAP = 640
_NG = _LCAP // 128
_PAD0 = _B
_OUTROWS = _B + _NW * 32

_BLK = 2048


def _gather_body(user_hbm, item_hbm, gu_t, gi_t, mu_t, mi_t,
                 gu_o, gi_o, mu_o, mi_o,
                 sbuf, lr_u, lk_u, lr_i, lk_i, br_u, bk_u, br_i, bk_i,
                 cb, cb2, rb, klist, k2d, cnt_u, cnt_i, scnt, sem, sem2):
    wid = lax.axis_index("s") * _NC + lax.axis_index("c")
    slab0 = wid * _SLAB
    hi = jnp.where(wid == _NW - 1, _TAILR, slab0 + _SLAB)
    iota = lax.iota(jnp.int32, 16)
    lane0m = iota == 0
    i32 = jnp.int32

    for ti, (idx_hbm, lr, lk) in enumerate(((user_hbm, lr_u, lk_u),
                                            (item_hbm, lr_i, lk_i))):
        scnt[ti] = 0
        for j0 in range(_B // 4096):
            pltpu.sync_copy(idx_hbm.at[pl.ds(j0 * 4096, 4096)], sbuf)

            def vloop(v, _):
                rv = sbuf[pl.ds(v * 16, 16)]
                for lane in range(16):
                    r = rv[lane]

                    @pl.when((r >= slab0) & (r < hi))
                    def _():
                        ct = jnp.minimum(scnt[ti], _LCAP - 1)
                        idx = jnp.where(
                            lane0m, jnp.full((16,), ct, i32),
                            _LCAP + iota)
                        k = (j0 * 4096) + v * 16 + lane
                        plsc.store_scatter(lr, [idx],
                                           jnp.full((16,), r, i32))
                        plsc.store_scatter(lk, [idx],
                                           jnp.full((16,), k, i32))
                        scnt[ti] = ct + 1
                return 0
            lax.fori_loop(0, 4096 // 16, vloop, 0)

    for ti, (lr, lk, br, bk, cnt) in enumerate(
            ((lr_u, lk_u, br_u, bk_u, cnt_u),
             (lr_i, lk_i, br_i, bk_i, cnt_i))):
        def zloop(c, _):
            cnt[c] = 0
            return 0
        lax.fori_loop(0, _NBUCK, zloop, 0)
        total = scnt[ti]

        def bloop(p, _):
            rv = lr[pl.ds(p * 16, 16)]
            kv = lk[pl.ds(p * 16, 16)]
            for lane in range(16):
                @pl.when(p * 16 + lane < total)
                def _():
                    r = rv[lane]
                    k = kv[lane]
                    c = jnp.minimum((r - slab0) >> 8, _NCHUNK - 1)
                    nc = jnp.minimum(cnt[c], _BCAP - 16)
                    base = c * _BCAP + nc
                    bidx = jnp.where(lane0m, jnp.full((16,), base, i32),
                                     _NBUCK * _BCAP + iota)
                    plsc.store_scatter(br, [bidx], jnp.full((16,), r, i32))
                    plsc.store_scatter(bk, [bidx], jnp.full((16,), k, i32))
                    cnt[c] = nc + 1
            return 0
        lax.fori_loop(0, _LCAP // 16, bloop, 0)

    for tab, br, bk, cnt, out in (
            (gu_t, br_u, bk_u, cnt_u, gu_o),
            (gi_t, br_i, bk_i, cnt_i, gi_o),
            (mu_t, br_u, bk_u, cnt_u, mu_o),
            (mi_t, br_i, bk_i, cnt_i, mi_o)):
        padbase = _PAD0 + wid * 32
        for j in range(_LCAP // 16):
            klist[pl.ds(j * 16, 16)] = padbase + iota + 16 * (j & 1)
        scnt[2] = 0

        def extract(c, lane0, buf):
            n_c = cnt[c]

            def gloop(g16, _):
                rv = br[pl.ds(c * _BCAP + g16 * 16, 16)]
                kv = bk[pl.ds(c * _BCAP + g16 * 16, 16)]
                for lane in range(16):
                    @pl.when(g16 * 16 + lane < n_c)
                    def _():
                        r = rv[lane]
                        k = kv[lane]
                        off = jnp.full((16,), r - lane0, i32)
                        g0 = plsc.load_gather(buf, [iota, off])
                        g1 = plsc.load_gather(buf, [iota + 16, off])
                        s = jnp.minimum(scnt[2], _LCAP - 1)
                        rb[s, pl.ds(0, 16)] = g0
                        rb[s, pl.ds(16, 16)] = g1
                        kidx = jnp.where(lane0m, jnp.full((16,), s, i32),
                                         _LCAP + iota)
                        plsc.store_scatter(klist, [kidx],
                                           jnp.full((16,), k, i32))
                        scnt[2] = s + 1
                return 0
            lax.fori_loop(0, (n_c + 15) >> 4, gloop, 0)

        def pairloop(g, _):
            c0 = 2 * g
            cpa = pltpu.make_async_copy(
                tab.at[:, pl.ds(slab0 + c0 * _CWL, _CWL)], cb, sem)
            cpb = pltpu.make_async_copy(
                tab.at[:, pl.ds(slab0 + (c0 + 1) * _CWL, _CWL)], cb2, sem2)
            cpa.start()
            cpb.start()
            cpa.wait()
            extract(c0, slab0 + c0 * _CWL, cb)
            cpb.wait()
            extract(c0 + 1, slab0 + (c0 + 1) * _CWL, cb2)
            return 0
        lax.fori_loop(0, _NCHUNK // 2, pairloop, 0)

        for g in range(_NG):
            for l in range(8):
                k2d[g, pl.ds(l * 16, 16)] = klist[pl.ds(g * 128 + l * 16, 16)]
        copies = [
            pltpu.async_copy(rb.at[pl.ds(g * 128, 128)],
                             out.at[k2d.at[g]], sem)
            for g in range(_NG)
        ]
        for cp in copies:
            cp.wait()


@functools.lru_cache(maxsize=None)
def _make_gather():
    return pl.kernel(
        _gather_body,
        out_type=[jax.ShapeDtypeStruct((_OUTROWS, 128), jnp.float32)] * 4,
        mesh=plsc.VectorSubcoreMesh(core_axis_name="c", subcore_axis_name="s"),
        compiler_params=pltpu.CompilerParams(needs_layout_passes=False),
        scratch_types=[
            pltpu.VMEM((4096,), jnp.int32),
            pltpu.VMEM((_LCAP + 16,), jnp.int32),
            pltpu.VMEM((_LCAP + 16,), jnp.int32),
            pltpu.VMEM((_LCAP + 16,), jnp.int32),
            pltpu.VMEM((_LCAP + 16,), jnp.int32),
            pltpu.VMEM((_NBUCK * _BCAP + 16,), jnp.int32),
            pltpu.VMEM((_NBUCK * _BCAP + 16,), jnp.int32),
            pltpu.VMEM((_NBUCK * _BCAP + 16,), jnp.int32),
            pltpu.VMEM((_NBUCK * _BCAP + 16,), jnp.int32),
            pltpu.VMEM((32, _CWL), jnp.float32),
            pltpu.VMEM((32, _CWL), jnp.float32),
            pltpu.VMEM((_LCAP, 128), jnp.float32),
            pltpu.VMEM((_LCAP + 16,), jnp.int32),
            pltpu.VMEM((_NG, 128), jnp.int32),
            pltpu.SMEM((_NBUCK,), jnp.int32),
            pltpu.SMEM((_NBUCK,), jnp.int32),
            pltpu.SMEM((4,), jnp.int32),
            pltpu.SemaphoreType.DMA,
            pltpu.SemaphoreType.DMA,
        ],
    )


def _mlp_body(gu, gi, mu, mi, uix, iix, gut, git, mut, mit,
              w0u, w0m, b0, w1, b1, w2, b2, wfg, wfm, bf, out):
    ui = uix[...]
    ii = iix[...]
    tail = jnp.arange(_TAILW, dtype=jnp.int32)[None, :]
    uoh = (ui[:, None] - _TAILR == tail).astype(jnp.float32)
    ioh = (ii[:, None] - _TAILR == tail).astype(jnp.float32)
    usel = (ui >= _TAILR).astype(jnp.float32)[:, None]
    isel = (ii >= _TAILR).astype(jnp.float32)[:, None]

    def fix(emb, oh, tl, sel):
        return emb[:, :_DIM] * (1.0 - sel) + (oh @ tl[...]) * sel

    mue = fix(mu, uoh, mut, usel)
    mie = fix(mi, ioh, mit, isel)
    h = jnp.maximum(mue @ w0u[...] + mie @ w0m[...] + b0[...], 0.0)
    h = jnp.maximum(h @ w1[...] + b1[...], 0.0)
    h = jnp.maximum(h @ w2[...] + b2[...], 0.0)
    g = fix(gu, uoh, gut, usel) * fix(gi, ioh, git, isel)
    out[...] = (jnp.sum(g * wfg[...], axis=1)
                + jnp.sum(h * wfm[...], axis=1) + bf[0, 0])


def _full(shape):
    return pl.BlockSpec(shape, lambda i: (0,) * len(shape))


@functools.lru_cache(maxsize=None)
def _make_head():
    return pl.pallas_call(
        _mlp_body,
        grid=(_B // _BLK,),
        in_specs=[pl.BlockSpec((_BLK, 128), lambda i: (i, 0))] * 4 + [
            pl.BlockSpec((_BLK,), lambda i: (i,)),
            pl.BlockSpec((_BLK,), lambda i: (i,)),
            _full((_TAILW, _DIM)),
            _full((_TAILW, _DIM)),
            _full((_TAILW, _DIM)),
            _full((_TAILW, _DIM)),
            _full((_DIM, 64)),
            _full((_DIM, 64)),
            _full((1, 64)),
            _full((64, 32)),
            _full((1, 32)),
            _full((32, 16)),
            _full((1, 16)),
            _full((1, _DIM)),
            _full((1, 16)),
            _full((1, 1)),
        ],
        out_specs=pl.BlockSpec((_BLK,), lambda i: (i,)),
        out_shape=jax.ShapeDtypeStruct((_B,), jnp.float32),
    )


def kernel(user, item, gmf_user_table, gmf_item_table, mlp_user_table,
           mlp_item_table, W0, b0, W1, b1, W2, b2, Wf, bf):
    user = user.astype(jnp.int32)
    item = item.astype(jnp.int32)
    gut = jnp.swapaxes(gmf_user_table, 0, 1)
    git = jnp.swapaxes(gmf_item_table, 0, 1)
    mut = jnp.swapaxes(mlp_user_table, 0, 1)
    mit = jnp.swapaxes(mlp_item_table, 0, 1)
    gu, gi, mu, mi = _make_gather()(user, item, gut, git, mut, mit)
    return _make_head()(
        gu, gi, mu, mi, user, item,
        gmf_user_table[_TAILR:], gmf_item_table[_TAILR:],
        mlp_user_table[_TAILR:], mlp_item_table[_TAILR:],
        W0[:_DIM], W0[_DIM:], b0.reshape(1, 64),
        W1, b1.reshape(1, 32), W2, b2.reshape(1, 16),
        Wf[:_DIM].reshape(1, _DIM), Wf[_DIM:].reshape(1, 16),
        bf.reshape(1, 1))


# speedup vs baseline: 1.8712x; 1.8712x over previous
import functools

import jax
import jax.numpy as jnp
from jax import lax
from jax.experimental import pallas as pl
from jax.experimental.pallas import tpu as pltpu
from jax.experimental.pallas import tpu_sc as plsc

_B = 16384
_DIM = 32
_N = 1000000
_NC = 2
_NS = 16
_NW = _NC * _NS
_SLAB = 244 * 128
_CWL = 256
_NCHUNK = 124
_TAILR = _NW * _SLAB + _CWL * 2
_TAILR = 999936
_TAILW = _N - _TAILR
_NBUCK = _NCHUNK
_BCAP = 40
_LCAP = 640
_NG = _LCAP // 128
_PAD0 = _B
_OUTROWS = _B + _NW * 32

_BLK = 2048


def _gather_body(user_hbm, item_hbm, gu_t, gi_t, mu_t, mi_t,
                 gu_o, gi_o, mu_o, mi_o,
                 sbuf, lr_u, lk_u, lr_i, lk_i, br_u, bk_u, br_i, bk_i,
                 cb, cb2, rb, klist, k2d, cnt_u, cnt_i, scnt, sem, sem2):
    wid = lax.axis_index("s") * _NC + lax.axis_index("c")
    slab0 = wid * _SLAB
    hi = jnp.where(wid == _NW - 1, _TAILR, slab0 + _SLAB)
    iota = lax.iota(jnp.int32, 16)
    lane0m = iota == 0
    i32 = jnp.int32

    for ti, (idx_hbm, lr, lk) in enumerate(((user_hbm, lr_u, lk_u),
                                            (item_hbm, lr_i, lk_i))):
        scnt[ti] = 0
        for j0 in range(_B // 4096):
            pltpu.sync_copy(idx_hbm.at[pl.ds(j0 * 4096, 4096)], sbuf)

            def vloop(v, _):
                rv = sbuf[pl.ds(v * 16, 16)]
                kv = iota + (j0 * 4096) + v * 16
                sl16 = jnp.full((16,), slab0, i32)
                hi16 = jnp.full((16,), hi, i32)
                m = (rv >= sl16) & (rv < hi16)
                ct = jnp.minimum(scnt[ti], _LCAP - 16)
                mi_ = m.astype(i32)
                incl = jnp.cumsum(mi_)
                idx = jnp.where(m, jnp.full((16,), ct, i32) + incl - mi_,
                                _LCAP + iota)
                plsc.store_scatter(lr, [idx], rv)
                plsc.store_scatter(lk, [idx], kv)
                scnt[ti] = ct + incl[15]
                return 0
            lax.fori_loop(0, 4096 // 16, vloop, 0)

    for ti, (lr, lk, br, bk, cnt) in enumerate(
            ((lr_u, lk_u, br_u, bk_u, cnt_u),
             (lr_i, lk_i, br_i, bk_i, cnt_i))):
        def zloop(c, _):
            cnt[c] = 0
            return 0
        lax.fori_loop(0, _NBUCK, zloop, 0)
        total = scnt[ti]

        def bloop(p, _):
            rv = lr[pl.ds(p * 16, 16)]
            kv = lk[pl.ds(p * 16, 16)]
            for lane in range(16):
                @pl.when(p * 16 + lane < total)
                def _():
                    r = rv[lane]
                    k = kv[lane]
                    c = jnp.minimum((r - slab0) >> 8, _NCHUNK - 1)
                    nc = jnp.minimum(cnt[c], _BCAP - 16)
                    base = c * _BCAP + nc
                    bidx = jnp.where(lane0m, jnp.full((16,), base, i32),
                                     _NBUCK * _BCAP + iota)
                    plsc.store_scatter(br, [bidx], jnp.full((16,), r, i32))
                    plsc.store_scatter(bk, [bidx], jnp.full((16,), k, i32))
                    cnt[c] = nc + 1
            return 0
        lax.fori_loop(0, _LCAP // 16, bloop, 0)

    for tab, br, bk, cnt, out in (
            (gu_t, br_u, bk_u, cnt_u, gu_o),
            (gi_t, br_i, bk_i, cnt_i, gi_o),
            (mu_t, br_u, bk_u, cnt_u, mu_o),
            (mi_t, br_i, bk_i, cnt_i, mi_o)):
        padbase = _PAD0 + wid * 32
        for j in range(_LCAP // 16):
            klist[pl.ds(j * 16, 16)] = padbase + iota + 16 * (j & 1)
        scnt[2] = 0

        def extract(c, lane0, buf):
            n_c = cnt[c]

            def gloop(g16, _):
                rv = br[pl.ds(c * _BCAP + g16 * 16, 16)]
                kv = bk[pl.ds(c * _BCAP + g16 * 16, 16)]
                for lane in range(16):
                    @pl.when(g16 * 16 + lane < n_c)
                    def _():
                        r = rv[lane]
                        k = kv[lane]
                        off = jnp.full((16,), r - lane0, i32)
                        g0 = plsc.load_gather(buf, [iota, off])
                        g1 = plsc.load_gather(buf, [iota + 16, off])
                        s = jnp.minimum(scnt[2], _LCAP - 1)
                        rb[s, pl.ds(0, 16)] = g0
                        rb[s, pl.ds(16, 16)] = g1
                        kidx = jnp.where(lane0m, jnp.full((16,), s, i32),
                                         _LCAP + iota)
                        plsc.store_scatter(klist, [kidx],
                                           jnp.full((16,), k, i32))
                        scnt[2] = s + 1
                return 0
            lax.fori_loop(0, (n_c + 15) >> 4, gloop, 0)

        def pairloop(g, _):
            c0 = 2 * g
            cpa = pltpu.make_async_copy(
                tab.at[:, pl.ds(slab0 + c0 * _CWL, _CWL)], cb, sem)
            cpb = pltpu.make_async_copy(
                tab.at[:, pl.ds(slab0 + (c0 + 1) * _CWL, _CWL)], cb2, sem2)
            cpa.start()
            cpb.start()
            cpa.wait()
            extract(c0, slab0 + c0 * _CWL, cb)
            cpb.wait()
            extract(c0 + 1, slab0 + (c0 + 1) * _CWL, cb2)
            return 0
        lax.fori_loop(0, _NCHUNK // 2, pairloop, 0)

        for g in range(_NG):
            for l in range(8):
                k2d[g, pl.ds(l * 16, 16)] = klist[pl.ds(g * 128 + l * 16, 16)]
        copies = [
            pltpu.async_copy(rb.at[pl.ds(g * 128, 128)],
                             out.at[k2d.at[g]], sem)
            for g in range(_NG)
        ]
        for cp in copies:
            cp.wait()


@functools.lru_cache(maxsize=None)
def _make_gather():
    return pl.kernel(
        _gather_body,
        out_type=[jax.ShapeDtypeStruct((_OUTROWS, 128), jnp.float32)] * 4,
        mesh=plsc.VectorSubcoreMesh(core_axis_name="c", subcore_axis_name="s"),
        compiler_params=pltpu.CompilerParams(needs_layout_passes=False),
        scratch_types=[
            pltpu.VMEM((4096,), jnp.int32),
            pltpu.VMEM((_LCAP + 16,), jnp.int32),
            pltpu.VMEM((_LCAP + 16,), jnp.int32),
            pltpu.VMEM((_LCAP + 16,), jnp.int32),
            pltpu.VMEM((_LCAP + 16,), jnp.int32),
            pltpu.VMEM((_NBUCK * _BCAP + 16,), jnp.int32),
            pltpu.VMEM((_NBUCK * _BCAP + 16,), jnp.int32),
            pltpu.VMEM((_NBUCK * _BCAP + 16,), jnp.int32),
            pltpu.VMEM((_NBUCK * _BCAP + 16,), jnp.int32),
            pltpu.VMEM((32, _CWL), jnp.float32),
            pltpu.VMEM((32, _CWL), jnp.float32),
            pltpu.VMEM((_LCAP, 128), jnp.float32),
            pltpu.VMEM((_LCAP + 16,), jnp.int32),
            pltpu.VMEM((_NG, 128), jnp.int32),
            pltpu.SMEM((_NBUCK,), jnp.int32),
            pltpu.SMEM((_NBUCK,), jnp.int32),
            pltpu.SMEM((4,), jnp.int32),
            pltpu.SemaphoreType.DMA,
            pltpu.SemaphoreType.DMA,
        ],
    )


def _mlp_body(gu, gi, mu, mi, uix, iix, gut, git, mut, mit,
              w0u, w0m, b0, w1, b1, w2, b2, wfg, wfm, bf, out):
    ui = uix[...]
    ii = iix[...]
    tail = jnp.arange(_TAILW, dtype=jnp.int32)[None, :]
    uoh = (ui[:, None] - _TAILR == tail).astype(jnp.float32)
    ioh = (ii[:, None] - _TAILR == tail).astype(jnp.float32)
    usel = (ui >= _TAILR).astype(jnp.float32)[:, None]
    isel = (ii >= _TAILR).astype(jnp.float32)[:, None]

    def fix(emb, oh, tl, sel):
        return emb[:, :_DIM] * (1.0 - sel) + (oh @ tl[...]) * sel

    mue = fix(mu, uoh, mut, usel)
    mie = fix(mi, ioh, mit, isel)
    h = jnp.maximum(mue @ w0u[...] + mie @ w0m[...] + b0[...], 0.0)
    h = jnp.maximum(h @ w1[...] + b1[...], 0.0)
    h = jnp.maximum(h @ w2[...] + b2[...], 0.0)
    g = fix(gu, uoh, gut, usel) * fix(gi, ioh, git, isel)
    out[...] = (jnp.sum(g * wfg[...], axis=1)
                + jnp.sum(h * wfm[...], axis=1) + bf[0, 0])


def _full(shape):
    return pl.BlockSpec(shape, lambda i: (0,) * len(shape))


@functools.lru_cache(maxsize=None)
def _make_head():
    return pl.pallas_call(
        _mlp_body,
        grid=(_B // _BLK,),
        in_specs=[pl.BlockSpec((_BLK, 128), lambda i: (i, 0))] * 4 + [
            pl.BlockSpec((_BLK,), lambda i: (i,)),
            pl.BlockSpec((_BLK,), lambda i: (i,)),
            _full((_TAILW, _DIM)),
            _full((_TAILW, _DIM)),
            _full((_TAILW, _DIM)),
            _full((_TAILW, _DIM)),
            _full((_DIM, 64)),
            _full((_DIM, 64)),
            _full((1, 64)),
            _full((64, 32)),
            _full((1, 32)),
            _full((32, 16)),
            _full((1, 16)),
            _full((1, _DIM)),
            _full((1, 16)),
            _full((1, 1)),
        ],
        out_specs=pl.BlockSpec((_BLK,), lambda i: (i,)),
        out_shape=jax.ShapeDtypeStruct((_B,), jnp.float32),
    )


def kernel(user, item, gmf_user_table, gmf_item_table, mlp_user_table,
           mlp_item_table, W0, b0, W1, b1, W2, b2, Wf, bf):
    user = user.astype(jnp.int32)
    item = item.astype(jnp.int32)
    gut = jnp.swapaxes(gmf_user_table, 0, 1)
    git = jnp.swapaxes(gmf_item_table, 0, 1)
    mut = jnp.swapaxes(mlp_user_table, 0, 1)
    mit = jnp.swapaxes(mlp_item_table, 0, 1)
    gu, gi, mu, mi = _make_gather()(user, item, gut, git, mut, mit)
    return _make_head()(
        gu, gi, mu, mi, user, item,
        gmf_user_table[_TAILR:], gmf_item_table[_TAILR:],
        mlp_user_table[_TAILR:], mlp_item_table[_TAILR:],
        W0[:_DIM], W0[_DIM:], b0.reshape(1, 64),
        W1, b1.reshape(1, 32), W2, b2.reshape(1, 16),
        Wf[:_DIM].reshape(1, _DIM), Wf[_DIM:].reshape(1, 16),
        bf.reshape(1, 1))


# speedup vs baseline: 1.8888x; 1.0094x over previous
import functools

import jax
import jax.numpy as jnp
from jax import lax
from jax.experimental import pallas as pl
from jax.experimental.pallas import tpu as pltpu
from jax.experimental.pallas import tpu_sc as plsc

_B = 16384
_DIM = 32
_N = 1000000
_NC = 2
_NS = 16
_NW = _NC * _NS
_SLAB = 244 * 128
_CWL = 256
_NCHUNK = 124
_TAILR = 999936
_TAILW = _N - _TAILR
_NBUCK = _NCHUNK
_BCAP = 40
_LCAP = 640
_NG = _LCAP // 128
_PAD0 = _B
_OUTROWS = _B + _NW * 32

_BLK = 2048


def _gather_body(user_hbm, item_hbm, gu_t, gi_t, mu_t, mi_t,
                 gu_o, gi_o, mu_o, mi_o,
                 sbuf, lr_u, lk_u, lr_i, lk_i, br_u, bk_u, br_i, bk_i,
                 cb, cb2, rb, klist, k2d, cnt_u, cnt_i, scnt, sem, sem2):
    wid = lax.axis_index("s") * _NC + lax.axis_index("c")
    slab0 = wid * _SLAB
    hi = jnp.where(wid == _NW - 1, _TAILR, slab0 + _SLAB)
    iota = lax.iota(jnp.int32, 16)
    lane0m = iota == 0
    i32 = jnp.int32

    for ti, (idx_hbm, lr, lk) in enumerate(((user_hbm, lr_u, lk_u),
                                            (item_hbm, lr_i, lk_i))):
        scnt[ti] = 0
        for j0 in range(_B // 4096):
            pltpu.sync_copy(idx_hbm.at[pl.ds(j0 * 4096, 4096)], sbuf)

            def vloop(v, _):
                rv = sbuf[pl.ds(v * 16, 16)]
                kv = iota + (j0 * 4096) + v * 16
                sl16 = jnp.full((16,), slab0, i32)
                hi16 = jnp.full((16,), hi, i32)
                m = (rv >= sl16) & (rv < hi16)
                ct = jnp.minimum(scnt[ti], _LCAP - 16)
                mi_ = m.astype(i32)
                incl = jnp.cumsum(mi_)
                idx = jnp.where(m, jnp.full((16,), ct, i32) + incl - mi_,
                                _LCAP + iota)
                plsc.store_scatter(lr, [idx], rv)
                plsc.store_scatter(lk, [idx], kv)
                scnt[ti] = ct + incl[15]
                return 0
            lax.fori_loop(0, 4096 // 16, vloop, 0)

    for ti, (lr, lk, br, bk, cnt) in enumerate(
            ((lr_u, lk_u, br_u, bk_u, cnt_u),
             (lr_i, lk_i, br_i, bk_i, cnt_i))):
        def zloop(c, _):
            cnt[c] = 0
            return 0
        lax.fori_loop(0, _NBUCK, zloop, 0)
        total = scnt[ti]

        def bloop(p, _):
            rv = lr[pl.ds(p * 16, 16)]
            kv = lk[pl.ds(p * 16, 16)]
            for lane in range(16):
                @pl.when(p * 16 + lane < total)
                def _():
                    r = rv[lane]
                    k = kv[lane]
                    c = jnp.minimum((r - slab0) >> 8, _NCHUNK - 1)
                    nc = jnp.minimum(cnt[c], _BCAP - 16)
                    base = c * _BCAP + nc
                    bidx = jnp.where(lane0m, jnp.full((16,), base, i32),
                                     _NBUCK * _BCAP + iota)
                    plsc.store_scatter(br, [bidx], jnp.full((16,), r, i32))
                    plsc.store_scatter(bk, [bidx], jnp.full((16,), k, i32))
                    cnt[c] = nc + 1
            return 0
        lax.fori_loop(0, _LCAP // 16, bloop, 0)

    for tab, br, bk, cnt, out in (
            (gu_t, br_u, bk_u, cnt_u, gu_o),
            (gi_t, br_i, bk_i, cnt_i, gi_o),
            (mu_t, br_u, bk_u, cnt_u, mu_o),
            (mi_t, br_i, bk_i, cnt_i, mi_o)):
        padbase = _PAD0 + wid * 32
        for j in range(_LCAP // 16):
            klist[pl.ds(j * 16, 16)] = padbase + iota + 16 * (j & 1)
        scnt[2] = 0

        def extract(c, lane0, buf):
            n_c = cnt[c]

            def gloop(g16, _):
                rv = br[pl.ds(c * _BCAP + g16 * 16, 16)]
                kv = bk[pl.ds(c * _BCAP + g16 * 16, 16)]
                for lane in range(16):
                    @pl.when(g16 * 16 + lane < n_c)
                    def _():
                        r = rv[lane]
                        k = kv[lane]
                        off = jnp.full((16,), r - lane0, i32)
                        g0 = plsc.load_gather(buf, [iota, off])
                        g1 = plsc.load_gather(buf, [iota + 16, off])
                        s = jnp.minimum(scnt[2], _LCAP - 1)
                        rb[s, pl.ds(0, 16)] = g0
                        rb[s, pl.ds(16, 16)] = g1
                        kidx = jnp.where(lane0m, jnp.full((16,), s, i32),
                                         _LCAP + iota)
                        plsc.store_scatter(klist, [kidx],
                                           jnp.full((16,), k, i32))
                        scnt[2] = s + 1
                return 0
            lax.fori_loop(0, (n_c + 15) >> 4, gloop, 0)

        def pairloop(g, _):
            c0 = 2 * g
            cpa = pltpu.make_async_copy(
                tab.at[:, pl.ds(slab0 + c0 * _CWL, _CWL)], cb, sem)
            cpb = pltpu.make_async_copy(
                tab.at[:, pl.ds(slab0 + (c0 + 1) * _CWL, _CWL)], cb2, sem2)
            cpa.start()
            cpb.start()
            cpa.wait()
            extract(c0, slab0 + c0 * _CWL, cb)
            cpb.wait()
            extract(c0 + 1, slab0 + (c0 + 1) * _CWL, cb2)
            return 0
        lax.fori_loop(0, _NCHUNK // 2, pairloop, 0)

        for g in range(_NG):
            for l in range(8):
                k2d[g, pl.ds(l * 16, 16)] = klist[pl.ds(g * 128 + l * 16, 16)]
        copies = [
            pltpu.async_copy(rb.at[pl.ds(g * 128, 128)],
                             out.at[k2d.at[g]], sem)
            for g in range(_NG)
        ]
        for cp in copies:
            cp.wait()


@functools.lru_cache(maxsize=None)
def _make_gather():
    return pl.kernel(
        _gather_body,
        out_type=[jax.ShapeDtypeStruct((_OUTROWS, 128), jnp.float32)] * 4,
        mesh=plsc.VectorSubcoreMesh(core_axis_name="c", subcore_axis_name="s"),
        compiler_params=pltpu.CompilerParams(needs_layout_passes=False),
        scratch_types=[
            pltpu.VMEM((4096,), jnp.int32),
            pltpu.VMEM((_LCAP + 16,), jnp.int32),
            pltpu.VMEM((_LCAP + 16,), jnp.int32),
            pltpu.VMEM((_LCAP + 16,), jnp.int32),
            pltpu.VMEM((_LCAP + 16,), jnp.int32),
            pltpu.VMEM((_NBUCK * _BCAP + 16,), jnp.int32),
            pltpu.VMEM((_NBUCK * _BCAP + 16,), jnp.int32),
            pltpu.VMEM((_NBUCK * _BCAP + 16,), jnp.int32),
            pltpu.VMEM((_NBUCK * _BCAP + 16,), jnp.int32),
            pltpu.VMEM((32, _CWL), jnp.float32),
            pltpu.VMEM((32, _CWL), jnp.float32),
            pltpu.VMEM((_LCAP, 128), jnp.float32),
            pltpu.VMEM((_LCAP + 16,), jnp.int32),
            pltpu.VMEM((_NG, 128), jnp.int32),
            pltpu.SMEM((_NBUCK,), jnp.int32),
            pltpu.SMEM((_NBUCK,), jnp.int32),
            pltpu.SMEM((4,), jnp.int32),
            pltpu.SemaphoreType.DMA,
            pltpu.SemaphoreType.DMA,
        ],
    )


def _mlp_body(gu, gi, mu, mi, uix, iix, gut, git, mut, mit,
              w0u, w0m, b0, w1, b1, w2, b2, wfg, wfm, bf, out):
    ui = uix[...]
    ii = iix[...]
    tail = jnp.arange(_TAILW, dtype=jnp.int32)[None, :]
    uoh = (ui[:, None] - _TAILR == tail).astype(jnp.float32)
    ioh = (ii[:, None] - _TAILR == tail).astype(jnp.float32)
    usel = (ui >= _TAILR).astype(jnp.float32)[:, None]
    isel = (ii >= _TAILR).astype(jnp.float32)[:, None]

    def fix(emb, oh, tl, sel):
        return emb[:, :_DIM] * (1.0 - sel) + (oh @ tl[...]) * sel

    mue = fix(mu, uoh, mut, usel)
    mie = fix(mi, ioh, mit, isel)
    h = jnp.maximum(mue @ w0u[...] + mie @ w0m[...] + b0[...], 0.0)
    h = jnp.maximum(h @ w1[...] + b1[...], 0.0)
    h = jnp.maximum(h @ w2[...] + b2[...], 0.0)
    g = fix(gu, uoh, gut, usel) * fix(gi, ioh, git, isel)
    out[...] = (jnp.sum(g * wfg[...], axis=1)
                + jnp.sum(h * wfm[...], axis=1) + bf[0, 0])


def _full(shape):
    return pl.BlockSpec(shape, lambda i: (0,) * len(shape))


@functools.lru_cache(maxsize=None)
def _make_head():
    return pl.pallas_call(
        _mlp_body,
        grid=(_B // _BLK,),
        in_specs=[pl.BlockSpec((_BLK, 128), lambda i: (i, 0))] * 4 + [
            pl.BlockSpec((_BLK,), lambda i: (i,)),
            pl.BlockSpec((_BLK,), lambda i: (i,)),
            _full((_TAILW, _DIM)),
            _full((_TAILW, _DIM)),
            _full((_TAILW, _DIM)),
            _full((_TAILW, _DIM)),
            _full((_DIM, 64)),
            _full((_DIM, 64)),
            _full((1, 64)),
            _full((64, 32)),
            _full((1, 32)),
            _full((32, 16)),
            _full((1, 16)),
            _full((1, _DIM)),
            _full((1, 16)),
            _full((1, 1)),
        ],
        out_specs=pl.BlockSpec((_BLK,), lambda i: (i,)),
        out_shape=jax.ShapeDtypeStruct((_B,), jnp.float32),
    )


def kernel(user, item, gmf_user_table, gmf_item_table, mlp_user_table,
           mlp_item_table, W0, b0, W1, b1, W2, b2, Wf, bf):
    user = user.astype(jnp.int32)
    item = item.astype(jnp.int32)
    gut = jnp.swapaxes(gmf_user_table, 0, 1)
    git = jnp.swapaxes(gmf_item_table, 0, 1)
    mut = jnp.swapaxes(mlp_user_table, 0, 1)
    mit = jnp.swapaxes(mlp_item_table, 0, 1)
    gu, gi, mu, mi = _make_gather()(user, item, gut, git, mut, mit)
    return _make_head()(
        gu, gi, mu, mi, user, item,
        gmf_user_table[_TAILR:], gmf_item_table[_TAILR:],
        mlp_user_table[_TAILR:], mlp_item_table[_TAILR:],
        W0[:_DIM], W0[_DIM:], b0.reshape(1, 64),
        W1, b1.reshape(1, 32), W2, b2.reshape(1, 16),
        Wf[:_DIM].reshape(1, _DIM), Wf[_DIM:].reshape(1, 16),
        bf.reshape(1, 1))


# speedup vs baseline: 2.1795x; 1.1539x over previous
import functools

import jax
import jax.numpy as jnp
from jax import lax
from jax.experimental import pallas as pl
from jax.experimental.pallas import tpu as pltpu
from jax.experimental.pallas import tpu_sc as plsc

_B = 16384
_DIM = 32
_N = 1000000
_NC = 2
_NS = 16
_NW = _NC * _NS
_SLAB = 244 * 128
_CWL = 256
_NCHUNK = 124
_TAILR = 999936
_TAILW = _N - _TAILR
_NBUCK = _NCHUNK
_BCAP = 40
_LCAP = 640
_NG = _LCAP // 128
_PAD0 = _B
_OUTROWS = _B + _NW * 32

_BLK = 2048


def _gather_body(user_hbm, item_hbm, gu_t, gi_t, mu_t, mi_t,
                 gu_o, gi_o, mu_o, mi_o,
                 sbuf, lr_u, lk_u, lr_i, lk_i, br_u, bk_u, br_i, bk_i,
                 cb, cb2, rb, klist, k2d, cnt_u, cnt_i, scnt, sem, sem2):
    wid = lax.axis_index("s") * _NC + lax.axis_index("c")
    slab0 = wid * _SLAB
    hi = jnp.where(wid == _NW - 1, _TAILR, slab0 + _SLAB)
    iota = lax.iota(jnp.int32, 16)
    lane0m = iota == 0
    i32 = jnp.int32

    for ti, (idx_hbm, lr, lk) in enumerate(((user_hbm, lr_u, lk_u),
                                            (item_hbm, lr_i, lk_i))):
        scnt[ti] = 0
        for j0 in range(_B // 4096):
            pltpu.sync_copy(idx_hbm.at[pl.ds(j0 * 4096, 4096)], sbuf)

            def vloop(v, _):
                rv = sbuf[pl.ds(v * 16, 16)]
                kv = iota + (j0 * 4096) + v * 16
                sl16 = jnp.full((16,), slab0, i32)
                hi16 = jnp.full((16,), hi, i32)
                m = (rv >= sl16) & (rv < hi16)
                ct = jnp.minimum(scnt[ti], _LCAP - 16)
                mi_ = m.astype(i32)
                incl = jnp.cumsum(mi_)
                idx = jnp.where(m, jnp.full((16,), ct, i32) + incl - mi_,
                                _LCAP + iota)
                plsc.store_scatter(lr, [idx], rv)
                plsc.store_scatter(lk, [idx], kv)
                scnt[ti] = ct + incl[15]
                return 0
            lax.fori_loop(0, 4096 // 16, vloop, 0)

    for ti, (lr, lk, br, bk, cnt) in enumerate(
            ((lr_u, lk_u, br_u, bk_u, cnt_u),
             (lr_i, lk_i, br_i, bk_i, cnt_i))):
        def zloop(c, _):
            cnt[c] = 0
            return 0
        lax.fori_loop(0, _NBUCK, zloop, 0)
        total = scnt[ti]

        def bloop(p, _):
            rv = lr[pl.ds(p * 16, 16)]
            kv = lk[pl.ds(p * 16, 16)]
            for lane in range(16):
                @pl.when(p * 16 + lane < total)
                def _():
                    r = rv[lane]
                    k = kv[lane]
                    c = jnp.minimum((r - slab0) >> 8, _NCHUNK - 1)
                    nc = jnp.minimum(cnt[c], _BCAP - 16)
                    base = c * _BCAP + nc
                    bidx = jnp.where(lane0m, jnp.full((16,), base, i32),
                                     _NBUCK * _BCAP + iota)
                    plsc.store_scatter(br, [bidx], jnp.full((16,), r, i32))
                    plsc.store_scatter(bk, [bidx], jnp.full((16,), k, i32))
                    cnt[c] = nc + 1
            return 0
        lax.fori_loop(0, _LCAP // 16, bloop, 0)

    for tab, br, bk, cnt, out in (
            (gu_t, br_u, bk_u, cnt_u, gu_o),
            (gi_t, br_i, bk_i, cnt_i, gi_o),
            (mu_t, br_u, bk_u, cnt_u, mu_o),
            (mi_t, br_i, bk_i, cnt_i, mi_o)):
        padbase = _PAD0 + wid * 32
        for j in range(_LCAP // 16):
            klist[pl.ds(j * 16, 16)] = padbase + iota + 16 * (j & 1)
        scnt[2] = 0

        def extract(c, lane0, buf):
            n_c = cnt[c]

            def gloop(g16, _):
                rv = br[pl.ds(c * _BCAP + g16 * 16, 16)]
                kv = bk[pl.ds(c * _BCAP + g16 * 16, 16)]
                for lane in range(16):
                    @pl.when(g16 * 16 + lane < n_c)
                    def _():
                        r = rv[lane]
                        k = kv[lane]
                        off = jnp.full((16,), r - lane0, i32)
                        g0 = plsc.load_gather(buf, [iota, off])
                        g1 = plsc.load_gather(buf, [iota + 16, off])
                        s = jnp.minimum(scnt[2], _LCAP - 1)
                        rb[s, pl.ds(0, 16)] = g0
                        rb[s, pl.ds(16, 16)] = g1
                        kidx = jnp.where(lane0m, jnp.full((16,), s, i32),
                                         _LCAP + iota)
                        plsc.store_scatter(klist, [kidx],
                                           jnp.full((16,), k, i32))
                        scnt[2] = s + 1
                return 0
            lax.fori_loop(0, (n_c + 15) >> 4, gloop, 0)

        def dma(c, buf, sm):
            return pltpu.make_async_copy(
                tab.at[:, pl.ds(slab0 + c * _CWL, _CWL)], buf, sm)

        dma(0, cb, sem).start()
        dma(1, cb2, sem2).start()

        def pairloop(g, _):
            c0 = 2 * g
            dma(c0, cb, sem).wait()
            extract(c0, slab0 + c0 * _CWL, cb)

            @pl.when(c0 + 2 < _NCHUNK)
            def _():
                dma(c0 + 2, cb, sem).start()
            dma(c0 + 1, cb2, sem2).wait()
            extract(c0 + 1, slab0 + (c0 + 1) * _CWL, cb2)

            @pl.when(c0 + 3 < _NCHUNK)
            def _():
                dma(c0 + 3, cb2, sem2).start()
            return 0
        lax.fori_loop(0, _NCHUNK // 2, pairloop, 0)

        for g in range(_NG):
            for l in range(8):
                k2d[g, pl.ds(l * 16, 16)] = klist[pl.ds(g * 128 + l * 16, 16)]
        copies = [
            pltpu.async_copy(rb.at[pl.ds(g * 128, 128)],
                             out.at[k2d.at[g]], sem)
            for g in range(_NG)
        ]
        for cp in copies:
            cp.wait()


@functools.lru_cache(maxsize=None)
def _make_gather():
    return pl.kernel(
        _gather_body,
        out_type=[jax.ShapeDtypeStruct((_OUTROWS, 128), jnp.float32)] * 4,
        mesh=plsc.VectorSubcoreMesh(core_axis_name="c", subcore_axis_name="s"),
        compiler_params=pltpu.CompilerParams(needs_layout_passes=False),
        scratch_types=[
            pltpu.VMEM((4096,), jnp.int32),
            pltpu.VMEM((_LCAP + 16,), jnp.int32),
            pltpu.VMEM((_LCAP + 16,), jnp.int32),
            pltpu.VMEM((_LCAP + 16,), jnp.int32),
            pltpu.VMEM((_LCAP + 16,), jnp.int32),
            pltpu.VMEM((_NBUCK * _BCAP + 16,), jnp.int32),
            pltpu.VMEM((_NBUCK * _BCAP + 16,), jnp.int32),
            pltpu.VMEM((_NBUCK * _BCAP + 16,), jnp.int32),
            pltpu.VMEM((_NBUCK * _BCAP + 16,), jnp.int32),
            pltpu.VMEM((32, _CWL), jnp.float32),
            pltpu.VMEM((32, _CWL), jnp.float32),
            pltpu.VMEM((_LCAP, 128), jnp.float32),
            pltpu.VMEM((_LCAP + 16,), jnp.int32),
            pltpu.VMEM((_NG, 128), jnp.int32),
            pltpu.SMEM((_NBUCK,), jnp.int32),
            pltpu.SMEM((_NBUCK,), jnp.int32),
            pltpu.SMEM((4,), jnp.int32),
            pltpu.SemaphoreType.DMA,
            pltpu.SemaphoreType.DMA,
        ],
    )


def _mlp_body(gu, gi, mu, mi, uix, iix, gut, git, mut, mit,
              w0u, w0m, b0, w1, b1, w2, b2, wfg, wfm, bf, out):
    ui = uix[...]
    ii = iix[...]
    tail = jnp.arange(_TAILW, dtype=jnp.int32)[None, :]
    uoh = (ui[:, None] - _TAILR == tail).astype(jnp.float32)
    ioh = (ii[:, None] - _TAILR == tail).astype(jnp.float32)
    usel = (ui >= _TAILR).astype(jnp.float32)[:, None]
    isel = (ii >= _TAILR).astype(jnp.float32)[:, None]

    def fix(emb, oh, tl, sel):
        return emb[:, :_DIM] * (1.0 - sel) + (oh @ tl[...]) * sel

    mue = fix(mu, uoh, mut, usel)
    mie = fix(mi, ioh, mit, isel)
    h = jnp.maximum(mue @ w0u[...] + mie @ w0m[...] + b0[...], 0.0)
    h = jnp.maximum(h @ w1[...] + b1[...], 0.0)
    h = jnp.maximum(h @ w2[...] + b2[...], 0.0)
    g = fix(gu, uoh, gut, usel) * fix(gi, ioh, git, isel)
    out[...] = (jnp.sum(g * wfg[...], axis=1)
                + jnp.sum(h * wfm[...], axis=1) + bf[0, 0])


def _full(shape):
    return pl.BlockSpec(shape, lambda i: (0,) * len(shape))


@functools.lru_cache(maxsize=None)
def _make_head():
    return pl.pallas_call(
        _mlp_body,
        grid=(_B // _BLK,),
        in_specs=[pl.BlockSpec((_BLK, 128), lambda i: (i, 0))] * 4 + [
            pl.BlockSpec((_BLK,), lambda i: (i,)),
            pl.BlockSpec((_BLK,), lambda i: (i,)),
            _full((_TAILW, _DIM)),
            _full((_TAILW, _DIM)),
            _full((_TAILW, _DIM)),
            _full((_TAILW, _DIM)),
            _full((_DIM, 64)),
            _full((_DIM, 64)),
            _full((1, 64)),
            _full((64, 32)),
            _full((1, 32)),
            _full((32, 16)),
            _full((1, 16)),
            _full((1, _DIM)),
            _full((1, 16)),
            _full((1, 1)),
        ],
        out_specs=pl.BlockSpec((_BLK,), lambda i: (i,)),
        out_shape=jax.ShapeDtypeStruct((_B,), jnp.float32),
    )


def kernel(user, item, gmf_user_table, gmf_item_table, mlp_user_table,
           mlp_item_table, W0, b0, W1, b1, W2, b2, Wf, bf):
    user = user.astype(jnp.int32)
    item = item.astype(jnp.int32)
    gut = jnp.swapaxes(gmf_user_table, 0, 1)
    git = jnp.swapaxes(gmf_item_table, 0, 1)
    mut = jnp.swapaxes(mlp_user_table, 0, 1)
    mit = jnp.swapaxes(mlp_item_table, 0, 1)
    gu, gi, mu, mi = _make_gather()(user, item, gut, git, mut, mit)
    return _make_head()(
        gu, gi, mu, mi, user, item,
        gmf_user_table[_TAILR:], gmf_item_table[_TAILR:],
        mlp_user_table[_TAILR:], mlp_item_table[_TAILR:],
        W0[:_DIM], W0[_DIM:], b0.reshape(1, 64),
        W1, b1.reshape(1, 32), W2, b2.reshape(1, 16),
        Wf[:_DIM].reshape(1, _DIM), Wf[_DIM:].reshape(1, 16),
        bf.reshape(1, 1))
